# Initial kernel scaffold; baseline (speedup 1.0000x reference)
import jax
import jax.numpy as jnp
from jax.experimental import pallas as pl


def kernel(X, theta):
    raise NotImplementedError("write your pallas kernel here")


# speedup vs baseline: 1.3810x; 1.3810x over previous
import functools

import jax
import jax.numpy as jnp
from jax import lax
from jax.experimental import pallas as pl
from jax.experimental.pallas import tpu as pltpu
from jax.experimental.pallas import tpu_sc as plsc

B, H, W, C = 4, 224, 224, 384
NW = 32
ROWS_PER_TILE = (B * H) // NW
K = 16
CHUNKS = W // K
NSLICE = C // 16

def _splat(v, dtype=jnp.int32):
    return jnp.full((16,), v, dtype)


def _bf16r(v):
    u = plsc.bitcast(v, jnp.int32)
    r = u + 0x7FFF + (jnp.right_shift(u, 16) & 1)
    r = r & jnp.int32(-65536)
    return plsc.bitcast(r, jnp.float32)


def _body(x_hbm, theta_hbm, grid_hbm, out_hbm, theta_v, grid_v, w_v,
          ga_v, gb_v, gc_v, gd_v, o_v, sem):
    wid = lax.axis_index("s") * 2 + lax.axis_index("c")
    pltpu.sync_copy(theta_hbm, theta_v)
    pltpu.sync_copy(grid_hbm, grid_v)

    def row_body(rr, _):
        r = wid * ROWS_PER_TILE + rr
        b = r // H
        i = r - b * H
        tb = b * 6
        t0 = _bf16r(plsc.load_gather(theta_v, [_splat(tb + 0)]))
        t1 = _bf16r(plsc.load_gather(theta_v, [_splat(tb + 1)]))
        t2 = _bf16r(plsc.load_gather(theta_v, [_splat(tb + 2)]))
        t3 = _bf16r(plsc.load_gather(theta_v, [_splat(tb + 3)]))
        t4 = _bf16r(plsc.load_gather(theta_v, [_splat(tb + 4)]))
        t5 = _bf16r(plsc.load_gather(theta_v, [_splat(tb + 5)]))
        gy = _bf16r(plsc.load_gather(grid_v, [_splat(W + i)]))
        cx = t1 * gy + t2
        cy = t4 * gy + t5
        base = b * (H * W)

        def chunk_body(c, _):
            gx = _bf16r(grid_v[pl.ds(c * K, 16)])
            x = t0 * gx + cx
            y = t3 * gx + cy
            px = 0.5 * (x + 1.0) * jnp.float32(W)
            py = 0.5 * (y + 1.0) * jnp.float32(H)
            xt = px.astype(jnp.int32)
            yt = py.astype(jnp.int32)
            x0 = jnp.where(xt.astype(jnp.float32) > px, xt - 1, xt)
            y0 = jnp.where(yt.astype(jnp.float32) > py, yt - 1, yt)
            x1 = x0 + 1
            y1 = y0 + 1
            x0 = jnp.clip(x0, 0, W - 1)
            x1 = jnp.clip(x1, 0, W - 1)
            y0 = jnp.clip(y0, 0, H - 1)
            y1 = jnp.clip(y1, 0, H - 1)
            x0f = x0.astype(jnp.float32)
            x1f = x1.astype(jnp.float32)
            y0f = y0.astype(jnp.float32)
            y1f = y1.astype(jnp.float32)
            w_v[pl.ds(0, 16)] = (x1f - px) * (y1f - py)
            w_v[pl.ds(16, 16)] = (x1f - px) * (py - y0f)
            w_v[pl.ds(32, 16)] = (px - x0f) * (y1f - py)
            w_v[pl.ds(48, 16)] = (px - x0f) * (py - y0f)
            row0 = y0 * W + base
            row1 = y1 * W + base
            cp_a = pltpu.async_copy(x_hbm.at[row0 + x0], ga_v, sem)
            cp_b = pltpu.async_copy(x_hbm.at[row1 + x0], gb_v, sem)
            cp_c = pltpu.async_copy(x_hbm.at[row0 + x1], gc_v, sem)
            cp_d = pltpu.async_copy(x_hbm.at[row1 + x1], gd_v, sem)
            cp_a.wait()
            cp_b.wait()
            cp_c.wait()
            cp_d.wait()

            def pix_body(p, _):
                pv = _splat(p)
                wa = plsc.load_gather(w_v, [pv])
                wb = plsc.load_gather(w_v, [pv + 16])
                wc = plsc.load_gather(w_v, [pv + 32])
                wd = plsc.load_gather(w_v, [pv + 48])
                for s in range(NSLICE):
                    sl = pl.ds(s * 16, 16)
                    o_v[p, sl] = (wa * ga_v[p, sl] + wb * gb_v[p, sl]
                                  + wc * gc_v[p, sl] + wd * gd_v[p, sl])
                return 0

            lax.fori_loop(0, K, pix_body, 0)
            start = r * W + c * K
            pltpu.sync_copy(o_v, out_hbm.at[pl.ds(start, K)])
            return 0

        lax.fori_loop(0, CHUNKS, chunk_body, 0)
        return 0

    lax.fori_loop(0, ROWS_PER_TILE, row_body, 0)


@jax.jit
def _sample(x_flat, theta_flat, grid):
    f = functools.partial(
        pl.kernel,
        out_type=jax.ShapeDtypeStruct((B * H * W, C), jnp.float32),
        mesh=plsc.VectorSubcoreMesh(core_axis_name="c", subcore_axis_name="s"),
        compiler_params=pltpu.CompilerParams(needs_layout_passes=False),
        scratch_types=[
            pltpu.VMEM((32,), jnp.float32),
            pltpu.VMEM((W + H,), jnp.float32),
            pltpu.VMEM((64,), jnp.float32),
            pltpu.VMEM((K, C), jnp.float32),
            pltpu.VMEM((K, C), jnp.float32),
            pltpu.VMEM((K, C), jnp.float32),
            pltpu.VMEM((K, C), jnp.float32),
            pltpu.VMEM((K, C), jnp.float32),
            pltpu.SemaphoreType.DMA,
        ],
    )(_body)
    return f(x_flat, theta_flat, grid)


def kernel(X, theta):
    x_flat = jnp.reshape(X, (B * H * W, C)).astype(jnp.float32)
    theta_flat = jnp.pad(jnp.reshape(theta, (-1,)).astype(jnp.float32), (0, 8))
    grid = jnp.concatenate(
        [jnp.linspace(-1.0, 1.0, W), jnp.linspace(-1.0, 1.0, H)]
    ).astype(jnp.float32)
    out = _sample(x_flat, theta_flat, grid)
    return jnp.reshape(out, (B, H, W, C))
